# baseline (speedup 1.0000x reference)
import functools

import jax
import jax.numpy as jnp
from jax import lax
from jax.experimental import pallas as pl
from jax.experimental.pallas import tpu as pltpu
from jax.experimental.pallas import tpu_sc as plsc

_B, _S, _D = 4, 8192, 768
_NC, _NS = 2, 16
_NW = _NC * _NS
_ROWS_W = _S // _NW
_NBUF = 4
_CH = 32
_CHUNKS = _ROWS_W // _CH

_mesh = plsc.VectorSubcoreMesh(core_axis_name="c", subcore_axis_name="s")


@functools.partial(
    pl.kernel,
    mesh=_mesh,
    out_type=jax.ShapeDtypeStruct((_B, _S, _D), jnp.float32),
    scratch_types=[pltpu.VMEM((_NBUF, _CH, _D), jnp.float32)]
    + [pltpu.SemaphoreType.DMA] * (2 * _NBUF),
)
def _sc_broadcast_copy(table_hbm, out_hbm, buf, *sems):
    rsems, wsems = sems[:_NBUF], sems[_NBUF:]
    wid = lax.axis_index("s") * _NC + lax.axis_index("c")
    base = wid * _ROWS_W
    writes = [[] for _ in range(_NBUF)]
    reads = [None] * _NBUF

    for i in range(min(_NBUF, _CHUNKS)):
        reads[i] = pltpu.async_copy(
            table_hbm.at[pl.ds(base + i * _CH, _CH)], buf.at[i], rsems[i]
        )
    for i in range(_CHUNKS):
        sl = i % _NBUF
        reads[sl].wait()
        r0 = base + i * _CH
        for b in range(_B):
            writes[sl].append(
                pltpu.async_copy(buf.at[sl], out_hbm.at[b, pl.ds(r0, _CH)], wsems[sl])
            )
        nxt = i + _NBUF
        if nxt < _CHUNKS:
            for w in writes[sl]:
                w.wait()
            writes[sl] = []
            reads[sl] = pltpu.async_copy(
                table_hbm.at[pl.ds(base + nxt * _CH, _CH)], buf.at[sl], rsems[sl]
            )
    for sl in range(_NBUF):
        for w in writes[sl]:
            w.wait()


def kernel(x, pos_embed_weight):
    del x
    return _sc_broadcast_copy(pos_embed_weight)
